# baseline (speedup 1.0000x reference)
import functools

import jax
import jax.numpy as jnp
from jax import lax
from jax.experimental import pallas as pl
from jax.experimental.pallas import tpu as pltpu
from jax.experimental.pallas import tpu_sc as plsc

_STRIDE = 4
_BS = 32
_NPTS = 17
_H = 128
_W = 128
_FLAT = _H * _W
_NCH = 2 * _NPTS
_PAD = 48
_NCHUNK = _FLAT // 16
_UNROLL = 8
_NBUF = 4

_T = 14
_TROWS = _T * _NPTS
_TBR = 34
_NB = _BS - _T
_R = _NB * _NPTS
_K = -(-_R // 32)
_NSLOT = 2 * _K
_NVEC = -(-_NSLOT // 16)
_OUTSZ = _BS * _PAD + 64
_DUMP = _BS * _PAD


def _tc_argmax_body(s_ref, o_ref):
    s = s_ref[0]
    m = jnp.max(s, axis=1, keepdims=True)
    iota = lax.broadcasted_iota(jnp.int32, (_TBR, _FLAT), 1)
    cand = jnp.where(s == m, iota, jnp.int32(_FLAT))
    idx = jnp.min(cand, axis=1, keepdims=True)
    o_ref[0] = jnp.broadcast_to(idx, (_TBR, 128))


def _tc_argmax(score_t):
    g = _TROWS // _TBR
    out = pl.pallas_call(
        _tc_argmax_body,
        grid=(g,),
        in_specs=[pl.BlockSpec((1, _TBR, _FLAT), lambda i: (i, 0, 0))],
        out_specs=pl.BlockSpec((1, _TBR, 128), lambda i: (i, 0, 0)),
        out_shape=jax.ShapeDtypeStruct((g, _TBR, 128), jnp.int32),
    )(score_t)
    return out.reshape(_TROWS * 128)


def _lane_shuffle(x, perm):
    return x.at[perm].get(mode="promise_in_bounds")


def _row_argmax(bufr, lane):
    def chunk_step(c, carry, bufr=bufr):
        rms, chs = carry
        cb = jnp.full((16,), c, jnp.int32)
        new_rms, new_chs = [], []
        for u in range(_UNROLL):
            v = bufr[pl.ds((c * _UNROLL + u) * 16, 16)]
            upd = v > rms[u]
            new_rms.append(jnp.where(upd, v, rms[u]))
            new_chs.append(jnp.where(upd, cb, chs[u]))
        return tuple(new_rms), tuple(new_chs)

    rm0 = jnp.full((16,), -jnp.inf, jnp.float32)
    rms, chs = lax.fori_loop(0, _NCHUNK // _UNROLL, chunk_step,
                             ((rm0,) * _UNROLL, (lane * 0,) * _UNROLL))
    m = rms[0]
    for u in range(1, _UNROLL):
        m = jnp.maximum(m, rms[u])
    for sh in (8, 4, 2, 1):
        m = jnp.maximum(m, _lane_shuffle(m, lane ^ sh))
    cand = jnp.full((16,), _FLAT, jnp.int32)
    for u in range(_UNROLL):
        flat_u = chs[u] * (16 * _UNROLL) + (u * 16) + lane
        cand = jnp.minimum(cand,
                           jnp.where(rms[u] == m, flat_u, jnp.int32(_FLAT)))
    for sh in (8, 4, 2, 1):
        cand = jnp.minimum(cand, _lane_shuffle(cand, lane ^ sh))
    return cand


def _sc_stream_body(score_hbm, off_hbm, out_hbm,
                    buf, addr_v, pos_v, vals_v, off_v,
                    sem0, sem1, sem2, sem3, semg):
    w = lax.axis_index("s") * 2 + lax.axis_index("c")
    lane = lax.iota(jnp.int32, 16)
    sems = (sem0, sem1, sem2, sem3)

    def row_off(k):
        rr = jnp.minimum(w + 32 * k, _R - 1)
        return (_TROWS + rr) * _FLAT

    handles = [None] * _K
    for k in range(min(_NBUF, _K)):
        handles[k] = pltpu.async_copy(
            score_hbm.at[pl.ds(row_off(k), _FLAT)], buf.at[k], sems[k])

    addrs = [lane * 0 for _ in range(_NVEC)]
    poss = [lane * 0 + _DUMP for _ in range(_NVEC)]
    coarses = [lane * 0 for _ in range(_NVEC)]
    for k in range(_K):
        handles[k].wait()
        cand = _row_argmax(buf.at[k % _NBUF], lane)
        if k + _NBUF < _K:
            handles[k + _NBUF] = pltpu.async_copy(
                score_hbm.at[pl.ds(row_off(k + _NBUF), _FLAT)],
                buf.at[k % _NBUF], sems[k % _NBUF])
        yv = cand >> 7
        xv = cand & (_W - 1)
        rr = w + 32 * k
        g = _TROWS + jnp.minimum(rr, _R - 1)
        b = g // _NPTS
        pt = g - b * _NPTS
        pad = rr >= _R
        addr0 = ((b * _NCH + 2 * pt) * _H + yv) * _W + xv
        p0 = jnp.where(pad, _DUMP + 2 * w, b * _PAD + 2 * pt)
        s0 = 2 * k
        vi, l0 = divmod(s0, 16)
        m0 = lane == l0
        m1 = lane == l0 + 1
        addrs[vi] = jnp.where(m0, addr0, addrs[vi])
        addrs[vi] = jnp.where(m1, addr0 + _FLAT, addrs[vi])
        pz = jnp.full((16,), 0, jnp.int32) + p0
        poss[vi] = jnp.where(m0, pz, poss[vi])
        poss[vi] = jnp.where(m1, pz + 1, poss[vi])
        coarses[vi] = jnp.where(m0, yv, coarses[vi])
        coarses[vi] = jnp.where(m1, xv, coarses[vi])
    for i in range(_NVEC):
        addr_v[pl.ds(16 * i, 16)] = addrs[i]
        pos_v[pl.ds(16 * i, 16)] = poss[i]
    pltpu.async_copy(off_hbm.at[addr_v], off_v, semg).wait()
    for i in range(_NVEC):
        off = off_v[pl.ds(16 * i, 16)]
        vals_v[pl.ds(16 * i, 16)] = (
            (coarses[i].astype(jnp.float32) + off) * float(_STRIDE))
    pltpu.async_copy(vals_v, out_hbm.at[pos_v], semg).wait()


def _sc_stream(score_flat, offset_flat):
    mesh = plsc.VectorSubcoreMesh(core_axis_name="c", subcore_axis_name="s")
    f = functools.partial(
        pl.kernel,
        mesh=mesh,
        out_type=jax.ShapeDtypeStruct((_OUTSZ,), jnp.float32),
        scratch_types=[
            pltpu.VMEM((_NBUF, _FLAT), jnp.float32),
            pltpu.VMEM((16 * _NVEC,), jnp.int32),
            pltpu.VMEM((16 * _NVEC,), jnp.int32),
            pltpu.VMEM((16 * _NVEC,), jnp.float32),
            pltpu.VMEM((16 * _NVEC,), jnp.float32),
            pltpu.SemaphoreType.DMA,
            pltpu.SemaphoreType.DMA,
            pltpu.SemaphoreType.DMA,
            pltpu.SemaphoreType.DMA,
            pltpu.SemaphoreType.DMA,
        ],
    )(_sc_stream_body)
    return f(score_flat, offset_flat)


def _sc_tail_body(idx_hbm, offtab_hbm, out_hbm,
                  ptrs_v, iv_v, offidx_v, off_v, out_v, sem):
    b = lax.axis_index("s") * 2 + lax.axis_index("c")
    bb = jnp.minimum(b, _T - 1)
    for base in (0, 16, 32):
        jv = lax.iota(jnp.int32, 16) + base
        ptrs_v[pl.ds(base, 16)] = (bb * _NPTS + (jv >> 1)) * 128
    pltpu.async_copy(idx_hbm.at[ptrs_v], iv_v, sem).wait()
    for base in (0, 16, 32):
        jv = lax.iota(jnp.int32, 16) + base
        iv = iv_v[pl.ds(base, 16)]
        yv = iv >> 7
        xv = iv & (_W - 1)
        oidx = ((bb * _NCH + jv) * _H + yv) * _W + xv
        offidx_v[pl.ds(base, 16)] = jnp.where(jv < _NCH, oidx, 0)
    pltpu.async_copy(offtab_hbm.at[offidx_v], off_v, sem).wait()
    for base in (0, 16, 32):
        jv = lax.iota(jnp.int32, 16) + base
        iv = iv_v[pl.ds(base, 16)]
        yv = iv >> 7
        xv = iv & (_W - 1)
        coarse = jnp.where((jv & 1) == 0, yv, xv).astype(jnp.float32)
        off = off_v[pl.ds(base, 16)]
        out_v[pl.ds(base, 16)] = (coarse + off) * float(_STRIDE)
    pltpu.sync_copy(out_v, out_hbm.at[pl.ds(bb * _PAD, _PAD)])


def _sc_tail(idx_flat, offset_flat):
    mesh = plsc.VectorSubcoreMesh(core_axis_name="c", subcore_axis_name="s")
    f = functools.partial(
        pl.kernel,
        mesh=mesh,
        out_type=jax.ShapeDtypeStruct((_T * _PAD,), jnp.float32),
        scratch_types=[
            pltpu.VMEM((_PAD,), jnp.int32),
            pltpu.VMEM((_PAD,), jnp.int32),
            pltpu.VMEM((_PAD,), jnp.int32),
            pltpu.VMEM((_PAD,), jnp.float32),
            pltpu.VMEM((_PAD,), jnp.float32),
            pltpu.SemaphoreType.DMA,
        ],
    )(_sc_tail_body)
    return f(idx_flat, offset_flat)


def kernel(score_map, offset_map):
    score_flat = score_map.reshape(_BS * _NPTS * _FLAT)
    offset_flat = offset_map.reshape(_BS * _NCH * _FLAT)
    out_sc = _sc_stream(score_flat, offset_flat)
    idx_tab = _tc_argmax(score_map.reshape(16, _TBR, _FLAT))
    out_tc = _sc_tail(idx_tab, offset_flat)
    full = jnp.concatenate(
        [out_tc.reshape(_T, _PAD),
         out_sc[_T * _PAD: _BS * _PAD].reshape(_NB, _PAD)], axis=0)
    return full[:, : _NCH].reshape(_BS, _NPTS, 2)


# speedup vs baseline: 1.7200x; 1.7200x over previous
import functools

import jax
import jax.numpy as jnp
from jax import lax
from jax.experimental import pallas as pl
from jax.experimental.pallas import tpu as pltpu
from jax.experimental.pallas import tpu_sc as plsc

_STRIDE = 4
_BS = 32
_NPTS = 17
_H = 128
_W = 128
_FLAT = _H * _W
_NCH = 2 * _NPTS
_PAD = 48
_NCHUNK = _FLAT // 16
_UNROLL = 8
_NBUF = 4

_T = 16
_TROWS = _T * _NPTS
_TBR = 34
_NB = _BS - _T
_K = 9
_NVEC = 2
_OUTSZ = _BS * _PAD


def _tc_argmax_body(s_ref, o_ref):
    s = s_ref[0]
    m = jnp.max(s, axis=1, keepdims=True)
    iota = lax.broadcasted_iota(jnp.int32, (_TBR, _FLAT), 1)
    cand = jnp.where(s == m, iota, jnp.int32(_FLAT))
    idx = jnp.min(cand, axis=1, keepdims=True)
    o_ref[0] = jnp.broadcast_to(idx, (_TBR, 128))


def _tc_argmax(score_t):
    g = _TROWS // _TBR
    out = pl.pallas_call(
        _tc_argmax_body,
        grid=(g,),
        in_specs=[pl.BlockSpec((1, _TBR, _FLAT), lambda i: (i, 0, 0))],
        out_specs=pl.BlockSpec((1, _TBR, 128), lambda i: (i, 0, 0)),
        out_shape=jax.ShapeDtypeStruct((g, _TBR, 128), jnp.int32),
    )(score_t)
    return out.reshape(_TROWS * 128)


def _lane_shuffle(x, perm):
    return x.at[perm].get(mode="promise_in_bounds")


def _row_argmax(bufr, lane):
    def chunk_step(c, carry, bufr=bufr):
        rms, chs = carry
        cb = jnp.full((16,), c, jnp.int32)
        new_rms, new_chs = [], []
        for u in range(_UNROLL):
            v = bufr[pl.ds((c * _UNROLL + u) * 16, 16)]
            upd = v > rms[u]
            new_rms.append(jnp.where(upd, v, rms[u]))
            new_chs.append(jnp.where(upd, cb, chs[u]))
        return tuple(new_rms), tuple(new_chs)

    rm0 = jnp.full((16,), -jnp.inf, jnp.float32)
    rms, chs = lax.fori_loop(0, _NCHUNK // _UNROLL, chunk_step,
                             ((rm0,) * _UNROLL, (lane * 0,) * _UNROLL))
    m = rms[0]
    for u in range(1, _UNROLL):
        m = jnp.maximum(m, rms[u])
    for sh in (8, 4, 2, 1):
        m = jnp.maximum(m, _lane_shuffle(m, lane ^ sh))
    cand = jnp.full((16,), _FLAT, jnp.int32)
    for u in range(_UNROLL):
        flat_u = chs[u] * (16 * _UNROLL) + (u * 16) + lane
        cand = jnp.minimum(cand,
                           jnp.where(rms[u] == m, flat_u, jnp.int32(_FLAT)))
    for sh in (8, 4, 2, 1):
        cand = jnp.minimum(cand, _lane_shuffle(cand, lane ^ sh))
    return cand


def _sc_stream_body(score_hbm, off_hbm, out_hbm,
                    buf, addr_v, vals_v, off_v,
                    sem0, sem1, sem2, sem3, semg):
    w = lax.axis_index("s") * 2 + lax.axis_index("c")
    lane = lax.iota(jnp.int32, 16)
    sems = (sem0, sem1, sem2, sem3)
    b = _T + (w >> 1)
    half = w & 1

    def row_off(k):
        return (b * _NPTS + half * 8 + k) * _FLAT

    handles = [None] * _K
    for k in range(_NBUF):
        handles[k] = pltpu.async_copy(
            score_hbm.at[pl.ds(row_off(k), _FLAT)], buf.at[k], sems[k])

    addrs = [lane * 0 for _ in range(_NVEC)]
    coarses = [lane * 0 for _ in range(_NVEC)]
    for k in range(_K):
        handles[k].wait()
        cand = _row_argmax(buf.at[k % _NBUF], lane)
        if k + _NBUF < _K:
            handles[k + _NBUF] = pltpu.async_copy(
                score_hbm.at[pl.ds(row_off(k + _NBUF), _FLAT)],
                buf.at[k % _NBUF], sems[k % _NBUF])
        yv = cand >> 7
        xv = cand & (_W - 1)
        pt = half * 8 + k
        addr0 = ((b * _NCH + 2 * pt) * _H + yv) * _W + xv
        vi, l0 = divmod(2 * k, 16)
        m0 = lane == l0
        m1 = lane == l0 + 1
        addrs[vi] = jnp.where(m0, addr0, addrs[vi])
        addrs[vi] = jnp.where(m1, addr0 + _FLAT, addrs[vi])
        coarses[vi] = jnp.where(m0, yv, coarses[vi])
        coarses[vi] = jnp.where(m1, xv, coarses[vi])
    for i in range(_NVEC):
        addr_v[pl.ds(16 * i, 16)] = addrs[i]
    pltpu.async_copy(off_hbm.at[addr_v], off_v, semg).wait()
    for i in range(_NVEC):
        off = off_v[pl.ds(16 * i, 16)]
        vals_v[pl.ds(16 * i, 16)] = (
            (coarses[i].astype(jnp.float32) + off) * float(_STRIDE))

    @pl.when(half == 0)
    def _():
        pltpu.sync_copy(vals_v.at[pl.ds(0, 16)],
                        out_hbm.at[pl.ds(b * _PAD, 16)])

    @pl.when(half == 1)
    def _():
        pltpu.sync_copy(vals_v,
                        out_hbm.at[pl.ds(b * _PAD + 16, 32)])


def _sc_stream(score_flat, offset_flat):
    mesh = plsc.VectorSubcoreMesh(core_axis_name="c", subcore_axis_name="s")
    f = functools.partial(
        pl.kernel,
        mesh=mesh,
        out_type=jax.ShapeDtypeStruct((_OUTSZ,), jnp.float32),
        scratch_types=[
            pltpu.VMEM((_NBUF, _FLAT), jnp.float32),
            pltpu.VMEM((16 * _NVEC,), jnp.int32),
            pltpu.VMEM((16 * _NVEC,), jnp.float32),
            pltpu.VMEM((16 * _NVEC,), jnp.float32),
            pltpu.SemaphoreType.DMA,
            pltpu.SemaphoreType.DMA,
            pltpu.SemaphoreType.DMA,
            pltpu.SemaphoreType.DMA,
            pltpu.SemaphoreType.DMA,
        ],
    )(_sc_stream_body)
    return f(score_flat, offset_flat)


def _sc_tail_body(idx_hbm, offtab_hbm, out_hbm,
                  ptrs_v, iv_v, offidx_v, off_v, out_v, sem):
    b = lax.axis_index("s") * 2 + lax.axis_index("c")
    bb = jnp.minimum(b, _T - 1)
    for base in (0, 16, 32):
        jv = lax.iota(jnp.int32, 16) + base
        ptrs_v[pl.ds(base, 16)] = (bb * _NPTS + (jv >> 1)) * 128
    pltpu.async_copy(idx_hbm.at[ptrs_v], iv_v, sem).wait()
    for base in (0, 16, 32):
        jv = lax.iota(jnp.int32, 16) + base
        iv = iv_v[pl.ds(base, 16)]
        yv = iv >> 7
        xv = iv & (_W - 1)
        oidx = ((bb * _NCH + jv) * _H + yv) * _W + xv
        offidx_v[pl.ds(base, 16)] = jnp.where(jv < _NCH, oidx, 0)
    pltpu.async_copy(offtab_hbm.at[offidx_v], off_v, sem).wait()
    for base in (0, 16, 32):
        jv = lax.iota(jnp.int32, 16) + base
        iv = iv_v[pl.ds(base, 16)]
        yv = iv >> 7
        xv = iv & (_W - 1)
        coarse = jnp.where((jv & 1) == 0, yv, xv).astype(jnp.float32)
        off = off_v[pl.ds(base, 16)]
        out_v[pl.ds(base, 16)] = (coarse + off) * float(_STRIDE)
    pltpu.sync_copy(out_v, out_hbm.at[pl.ds(bb * _PAD, _PAD)])


def _sc_tail(idx_flat, offset_flat):
    mesh = plsc.VectorSubcoreMesh(core_axis_name="c", subcore_axis_name="s")
    f = functools.partial(
        pl.kernel,
        mesh=mesh,
        out_type=jax.ShapeDtypeStruct((_T * _PAD,), jnp.float32),
        scratch_types=[
            pltpu.VMEM((_PAD,), jnp.int32),
            pltpu.VMEM((_PAD,), jnp.int32),
            pltpu.VMEM((_PAD,), jnp.int32),
            pltpu.VMEM((_PAD,), jnp.float32),
            pltpu.VMEM((_PAD,), jnp.float32),
            pltpu.SemaphoreType.DMA,
        ],
    )(_sc_tail_body)
    return f(idx_flat, offset_flat)


def kernel(score_map, offset_map):
    score_flat = score_map.reshape(_BS * _NPTS * _FLAT)
    offset_flat = offset_map.reshape(_BS * _NCH * _FLAT)
    out_sc = _sc_stream(score_flat, offset_flat)
    idx_tab = _tc_argmax(score_map.reshape(16, _TBR, _FLAT))
    out_tc = _sc_tail(idx_tab, offset_flat)
    full = jnp.concatenate(
        [out_tc.reshape(_T, _PAD),
         out_sc[_T * _PAD:].reshape(_NB, _PAD)], axis=0)
    return full[:, : _NCH].reshape(_BS, _NPTS, 2)


# speedup vs baseline: 1.7257x; 1.0033x over previous
import functools

import jax
import jax.numpy as jnp
from jax import lax
from jax.experimental import pallas as pl
from jax.experimental.pallas import tpu as pltpu
from jax.experimental.pallas import tpu_sc as plsc

_STRIDE = 4
_BS = 32
_NPTS = 17
_H = 128
_W = 128
_FLAT = _H * _W
_NCH = 2 * _NPTS
_PAD = 48
_NCHUNK = _FLAT // 16
_UNROLL = 8
_NBUF = 4

_T = 16
_TROWS = _T * _NPTS
_TBR = 34
_NB = _BS - _T
_K = 9
_NVEC = 2
_OUTSZ = _BS * _PAD


def _tc_argmax_body(s_ref, o_ref):
    s = s_ref[0]
    m = jnp.max(s, axis=1, keepdims=True)
    iota = lax.broadcasted_iota(jnp.int32, (_TBR, _FLAT), 1)
    cand = jnp.where(s == m, iota, jnp.int32(_FLAT))
    idx = jnp.min(cand, axis=1, keepdims=True)
    o_ref[0] = jnp.broadcast_to(idx, (_TBR, 128))


def _tc_argmax(score_t):
    g = _TROWS // _TBR
    out = pl.pallas_call(
        _tc_argmax_body,
        grid=(g,),
        in_specs=[pl.BlockSpec((1, _TBR, _FLAT), lambda i: (i, 0, 0))],
        out_specs=pl.BlockSpec((1, _TBR, 128), lambda i: (i, 0, 0)),
        out_shape=jax.ShapeDtypeStruct((g, _TBR, 128), jnp.int32),
    )(score_t)
    return out.reshape(_TROWS * 128)


def _lane_shuffle(x, perm):
    return x.at[perm].get(mode="promise_in_bounds")


def _row_argmax(bufr, lane):
    def chunk_step(c, carry, bufr=bufr):
        rms, chs = carry
        cb = jnp.full((16,), c, jnp.int32)
        new_rms, new_chs = [], []
        for u in range(_UNROLL):
            v = bufr[pl.ds((c * _UNROLL + u) * 16, 16)]
            upd = v > rms[u]
            new_rms.append(jnp.where(upd, v, rms[u]))
            new_chs.append(jnp.where(upd, cb, chs[u]))
        return tuple(new_rms), tuple(new_chs)

    rm0 = jnp.full((16,), -jnp.inf, jnp.float32)
    rms, chs = lax.fori_loop(0, _NCHUNK // _UNROLL, chunk_step,
                             ((rm0,) * _UNROLL, (lane * 0,) * _UNROLL))
    m = rms[0]
    for u in range(1, _UNROLL):
        m = jnp.maximum(m, rms[u])
    for sh in (8, 4, 2, 1):
        m = jnp.maximum(m, _lane_shuffle(m, lane ^ sh))
    cand = jnp.full((16,), _FLAT, jnp.int32)
    for u in range(_UNROLL):
        flat_u = chs[u] * (16 * _UNROLL) + (u * 16) + lane
        cand = jnp.minimum(cand,
                           jnp.where(rms[u] == m, flat_u, jnp.int32(_FLAT)))
    for sh in (8, 4, 2, 1):
        cand = jnp.minimum(cand, _lane_shuffle(cand, lane ^ sh))
    return cand


def _sc_stream_body(score_hbm, off_hbm, out_hbm,
                    buf, cands_v, addr_v, vals_v, off_v, sem, semg):
    w = lax.axis_index("s") * 2 + lax.axis_index("c")
    lane = lax.iota(jnp.int32, 16)
    b = _T + (w >> 1)
    half = w & 1

    def row_off(k):
        return (b * _NPTS + half * 8 + k) * _FLAT

    for k in range(_NBUF):
        pltpu.async_copy(
            score_hbm.at[pl.ds(row_off(k), _FLAT)], buf.at[k], sem)

    def row_step(k, _):
        slot = k & (_NBUF - 1)
        pltpu.make_async_copy(
            score_hbm.at[pl.ds(row_off(k), _FLAT)], buf.at[slot], sem).wait()
        cand = _row_argmax(buf.at[slot], lane)
        cands_v[pl.ds(k * 16, 16)] = cand

        @pl.when(k + _NBUF < _K)
        def _():
            pltpu.async_copy(
                score_hbm.at[pl.ds(row_off(k + _NBUF), _FLAT)],
                buf.at[slot], sem)
        return 0

    lax.fori_loop(0, _K, row_step, 0)

    addrs = [lane * 0 for _ in range(_NVEC)]
    coarses = [lane * 0 for _ in range(_NVEC)]
    for k in range(_K):
        cand = cands_v[pl.ds(k * 16, 16)]
        yv = cand >> 7
        xv = cand & (_W - 1)
        pt = half * 8 + k
        addr0 = ((b * _NCH + 2 * pt) * _H + yv) * _W + xv
        vi, l0 = divmod(2 * k, 16)
        m0 = lane == l0
        m1 = lane == l0 + 1
        addrs[vi] = jnp.where(m0, addr0, addrs[vi])
        addrs[vi] = jnp.where(m1, addr0 + _FLAT, addrs[vi])
        coarses[vi] = jnp.where(m0, yv, coarses[vi])
        coarses[vi] = jnp.where(m1, xv, coarses[vi])
    for i in range(_NVEC):
        addr_v[pl.ds(16 * i, 16)] = addrs[i]
    pltpu.async_copy(off_hbm.at[addr_v], off_v, semg).wait()
    for i in range(_NVEC):
        off = off_v[pl.ds(16 * i, 16)]
        vals_v[pl.ds(16 * i, 16)] = (
            (coarses[i].astype(jnp.float32) + off) * float(_STRIDE))

    @pl.when(half == 0)
    def _():
        pltpu.sync_copy(vals_v.at[pl.ds(0, 16)],
                        out_hbm.at[pl.ds(b * _PAD, 16)])

    @pl.when(half == 1)
    def _():
        pltpu.sync_copy(vals_v,
                        out_hbm.at[pl.ds(b * _PAD + 16, 32)])


def _sc_stream(score_flat, offset_flat):
    mesh = plsc.VectorSubcoreMesh(core_axis_name="c", subcore_axis_name="s")
    f = functools.partial(
        pl.kernel,
        mesh=mesh,
        out_type=jax.ShapeDtypeStruct((_OUTSZ,), jnp.float32),
        scratch_types=[
            pltpu.VMEM((_NBUF, _FLAT), jnp.float32),
            pltpu.VMEM((16 * _K,), jnp.int32),
            pltpu.VMEM((16 * _NVEC,), jnp.int32),
            pltpu.VMEM((16 * _NVEC,), jnp.float32),
            pltpu.VMEM((16 * _NVEC,), jnp.float32),
            pltpu.SemaphoreType.DMA,
            pltpu.SemaphoreType.DMA,
        ],
    )(_sc_stream_body)
    return f(score_flat, offset_flat)


def _sc_tail_body(idx_hbm, offtab_hbm, out_hbm,
                  ptrs_v, iv_v, offidx_v, off_v, out_v, sem):
    b = lax.axis_index("s") * 2 + lax.axis_index("c")
    bb = jnp.minimum(b, _T - 1)
    for base in (0, 16, 32):
        jv = lax.iota(jnp.int32, 16) + base
        ptrs_v[pl.ds(base, 16)] = (bb * _NPTS + (jv >> 1)) * 128
    pltpu.async_copy(idx_hbm.at[ptrs_v], iv_v, sem).wait()
    for base in (0, 16, 32):
        jv = lax.iota(jnp.int32, 16) + base
        iv = iv_v[pl.ds(base, 16)]
        yv = iv >> 7
        xv = iv & (_W - 1)
        oidx = ((bb * _NCH + jv) * _H + yv) * _W + xv
        offidx_v[pl.ds(base, 16)] = jnp.where(jv < _NCH, oidx, 0)
    pltpu.async_copy(offtab_hbm.at[offidx_v], off_v, sem).wait()
    for base in (0, 16, 32):
        jv = lax.iota(jnp.int32, 16) + base
        iv = iv_v[pl.ds(base, 16)]
        yv = iv >> 7
        xv = iv & (_W - 1)
        coarse = jnp.where((jv & 1) == 0, yv, xv).astype(jnp.float32)
        off = off_v[pl.ds(base, 16)]
        out_v[pl.ds(base, 16)] = (coarse + off) * float(_STRIDE)
    pltpu.sync_copy(out_v, out_hbm.at[pl.ds(bb * _PAD, _PAD)])


def _sc_tail(idx_flat, offset_flat):
    mesh = plsc.VectorSubcoreMesh(core_axis_name="c", subcore_axis_name="s")
    f = functools.partial(
        pl.kernel,
        mesh=mesh,
        out_type=jax.ShapeDtypeStruct((_T * _PAD,), jnp.float32),
        scratch_types=[
            pltpu.VMEM((_PAD,), jnp.int32),
            pltpu.VMEM((_PAD,), jnp.int32),
            pltpu.VMEM((_PAD,), jnp.int32),
            pltpu.VMEM((_PAD,), jnp.float32),
            pltpu.VMEM((_PAD,), jnp.float32),
            pltpu.SemaphoreType.DMA,
        ],
    )(_sc_tail_body)
    return f(idx_flat, offset_flat)


def kernel(score_map, offset_map):
    score_flat = score_map.reshape(_BS * _NPTS * _FLAT)
    offset_flat = offset_map.reshape(_BS * _NCH * _FLAT)
    out_sc = _sc_stream(score_flat, offset_flat)
    idx_tab = _tc_argmax(score_map.reshape(16, _TBR, _FLAT))
    out_tc = _sc_tail(idx_tab, offset_flat)
    full = jnp.concatenate(
        [out_tc.reshape(_T, _PAD),
         out_sc[_T * _PAD:].reshape(_NB, _PAD)], axis=0)
    return full[:, : _NCH].reshape(_BS, _NPTS, 2)


# speedup vs baseline: 1.7266x; 1.0005x over previous
import functools

import jax
import jax.numpy as jnp
from jax import lax
from jax.experimental import pallas as pl
from jax.experimental.pallas import tpu as pltpu
from jax.experimental.pallas import tpu_sc as plsc

_STRIDE = 4
_BS = 32
_NPTS = 17
_H = 128
_W = 128
_FLAT = _H * _W
_NCH = 2 * _NPTS
_PAD = 48
_NCHUNK = _FLAT // 16
_UNROLL = 8
_NBUF = 2

_T = 16
_TROWS = _T * _NPTS
_TBR = 34
_NB = _BS - _T
_K = 9
_NVEC = 2
_OUTSZ = _BS * _PAD


def _tc_argmax_body(s_ref, o_ref):
    s = s_ref[0]
    m = jnp.max(s, axis=1, keepdims=True)
    iota = lax.broadcasted_iota(jnp.int32, (_TBR, _FLAT), 1)
    cand = jnp.where(s == m, iota, jnp.int32(_FLAT))
    idx = jnp.min(cand, axis=1, keepdims=True)
    o_ref[0] = jnp.broadcast_to(idx, (_TBR, 128))


def _tc_argmax(score_t):
    g = _TROWS // _TBR
    out = pl.pallas_call(
        _tc_argmax_body,
        grid=(g,),
        in_specs=[pl.BlockSpec((1, _TBR, _FLAT), lambda i: (i, 0, 0))],
        out_specs=pl.BlockSpec((1, _TBR, 128), lambda i: (i, 0, 0)),
        out_shape=jax.ShapeDtypeStruct((g, _TBR, 128), jnp.int32),
    )(score_t)
    return out.reshape(_TROWS * 128)


def _lane_shuffle(x, perm):
    return x.at[perm].get(mode="promise_in_bounds")


def _row_argmax(bufr, lane):
    def chunk_step(c, carry, bufr=bufr):
        rms, chs = carry
        cb = jnp.full((16,), c, jnp.int32)
        new_rms, new_chs = [], []
        for u in range(_UNROLL):
            v = bufr[pl.ds((c * _UNROLL + u) * 16, 16)]
            upd = v > rms[u]
            new_rms.append(jnp.where(upd, v, rms[u]))
            new_chs.append(jnp.where(upd, cb, chs[u]))
        return tuple(new_rms), tuple(new_chs)

    rm0 = jnp.full((16,), -jnp.inf, jnp.float32)
    rms, chs = lax.fori_loop(0, _NCHUNK // _UNROLL, chunk_step,
                             ((rm0,) * _UNROLL, (lane * 0,) * _UNROLL))
    m = rms[0]
    for u in range(1, _UNROLL):
        m = jnp.maximum(m, rms[u])
    for sh in (8, 4, 2, 1):
        m = jnp.maximum(m, _lane_shuffle(m, lane ^ sh))
    cand = jnp.full((16,), _FLAT, jnp.int32)
    for u in range(_UNROLL):
        flat_u = chs[u] * (16 * _UNROLL) + (u * 16) + lane
        cand = jnp.minimum(cand,
                           jnp.where(rms[u] == m, flat_u, jnp.int32(_FLAT)))
    for sh in (8, 4, 2, 1):
        cand = jnp.minimum(cand, _lane_shuffle(cand, lane ^ sh))
    return cand


def _sc_stream_body(score_hbm, off_hbm, out_hbm,
                    buf, cands_v, addr_v, vals_v, off_v, sem, semg):
    w = lax.axis_index("s") * 2 + lax.axis_index("c")
    lane = lax.iota(jnp.int32, 16)
    b = _T + (w >> 1)
    half = w & 1

    def row_off(k):
        return (b * _NPTS + half * 8 + k) * _FLAT

    for k in range(_NBUF):
        pltpu.async_copy(
            score_hbm.at[pl.ds(row_off(k), _FLAT)], buf.at[k], sem)

    def row_step(k, _):
        slot = k & (_NBUF - 1)
        pltpu.make_async_copy(
            score_hbm.at[pl.ds(row_off(k), _FLAT)], buf.at[slot], sem).wait()
        cand = _row_argmax(buf.at[slot], lane)
        cands_v[pl.ds(k * 16, 16)] = cand

        @pl.when(k + _NBUF < _K)
        def _():
            pltpu.async_copy(
                score_hbm.at[pl.ds(row_off(k + _NBUF), _FLAT)],
                buf.at[slot], sem)
        return 0

    lax.fori_loop(0, _K, row_step, 0)

    addrs = [lane * 0 for _ in range(_NVEC)]
    coarses = [lane * 0 for _ in range(_NVEC)]
    for k in range(_K):
        cand = cands_v[pl.ds(k * 16, 16)]
        yv = cand >> 7
        xv = cand & (_W - 1)
        pt = half * 8 + k
        addr0 = ((b * _NCH + 2 * pt) * _H + yv) * _W + xv
        vi, l0 = divmod(2 * k, 16)
        m0 = lane == l0
        m1 = lane == l0 + 1
        addrs[vi] = jnp.where(m0, addr0, addrs[vi])
        addrs[vi] = jnp.where(m1, addr0 + _FLAT, addrs[vi])
        coarses[vi] = jnp.where(m0, yv, coarses[vi])
        coarses[vi] = jnp.where(m1, xv, coarses[vi])
    for i in range(_NVEC):
        addr_v[pl.ds(16 * i, 16)] = addrs[i]
    pltpu.async_copy(off_hbm.at[addr_v], off_v, semg).wait()
    for i in range(_NVEC):
        off = off_v[pl.ds(16 * i, 16)]
        vals_v[pl.ds(16 * i, 16)] = (
            (coarses[i].astype(jnp.float32) + off) * float(_STRIDE))

    @pl.when(half == 0)
    def _():
        pltpu.sync_copy(vals_v.at[pl.ds(0, 16)],
                        out_hbm.at[pl.ds(b * _PAD, 16)])

    @pl.when(half == 1)
    def _():
        pltpu.sync_copy(vals_v,
                        out_hbm.at[pl.ds(b * _PAD + 16, 32)])


def _sc_stream(score_flat, offset_flat):
    mesh = plsc.VectorSubcoreMesh(core_axis_name="c", subcore_axis_name="s")
    f = functools.partial(
        pl.kernel,
        mesh=mesh,
        out_type=jax.ShapeDtypeStruct((_OUTSZ,), jnp.float32),
        scratch_types=[
            pltpu.VMEM((_NBUF, _FLAT), jnp.float32),
            pltpu.VMEM((16 * _K,), jnp.int32),
            pltpu.VMEM((16 * _NVEC,), jnp.int32),
            pltpu.VMEM((16 * _NVEC,), jnp.float32),
            pltpu.VMEM((16 * _NVEC,), jnp.float32),
            pltpu.SemaphoreType.DMA,
            pltpu.SemaphoreType.DMA,
        ],
    )(_sc_stream_body)
    return f(score_flat, offset_flat)


def _sc_tail_body(idx_hbm, offtab_hbm, out_hbm,
                  ptrs_v, iv_v, offidx_v, off_v, out_v, sem):
    b = lax.axis_index("s") * 2 + lax.axis_index("c")
    bb = jnp.minimum(b, _T - 1)
    for base in (0, 16, 32):
        jv = lax.iota(jnp.int32, 16) + base
        ptrs_v[pl.ds(base, 16)] = (bb * _NPTS + (jv >> 1)) * 128
    pltpu.async_copy(idx_hbm.at[ptrs_v], iv_v, sem).wait()
    for base in (0, 16, 32):
        jv = lax.iota(jnp.int32, 16) + base
        iv = iv_v[pl.ds(base, 16)]
        yv = iv >> 7
        xv = iv & (_W - 1)
        oidx = ((bb * _NCH + jv) * _H + yv) * _W + xv
        offidx_v[pl.ds(base, 16)] = jnp.where(jv < _NCH, oidx, 0)
    pltpu.async_copy(offtab_hbm.at[offidx_v], off_v, sem).wait()
    for base in (0, 16, 32):
        jv = lax.iota(jnp.int32, 16) + base
        iv = iv_v[pl.ds(base, 16)]
        yv = iv >> 7
        xv = iv & (_W - 1)
        coarse = jnp.where((jv & 1) == 0, yv, xv).astype(jnp.float32)
        off = off_v[pl.ds(base, 16)]
        out_v[pl.ds(base, 16)] = (coarse + off) * float(_STRIDE)
    pltpu.sync_copy(out_v, out_hbm.at[pl.ds(bb * _PAD, _PAD)])


def _sc_tail(idx_flat, offset_flat):
    mesh = plsc.VectorSubcoreMesh(core_axis_name="c", subcore_axis_name="s")
    f = functools.partial(
        pl.kernel,
        mesh=mesh,
        out_type=jax.ShapeDtypeStruct((_T * _PAD,), jnp.float32),
        scratch_types=[
            pltpu.VMEM((_PAD,), jnp.int32),
            pltpu.VMEM((_PAD,), jnp.int32),
            pltpu.VMEM((_PAD,), jnp.int32),
            pltpu.VMEM((_PAD,), jnp.float32),
            pltpu.VMEM((_PAD,), jnp.float32),
            pltpu.SemaphoreType.DMA,
        ],
    )(_sc_tail_body)
    return f(idx_flat, offset_flat)


def kernel(score_map, offset_map):
    score_flat = score_map.reshape(_BS * _NPTS * _FLAT)
    offset_flat = offset_map.reshape(_BS * _NCH * _FLAT)
    out_sc = _sc_stream(score_flat, offset_flat)
    idx_tab = _tc_argmax(score_map.reshape(16, _TBR, _FLAT))
    out_tc = _sc_tail(idx_tab, offset_flat)
    full = jnp.concatenate(
        [out_tc.reshape(_T, _PAD),
         out_sc[_T * _PAD:].reshape(_NB, _PAD)], axis=0)
    return full[:, : _NCH].reshape(_BS, _NPTS, 2)


# speedup vs baseline: 2.5610x; 1.4833x over previous
import functools

import jax
import jax.numpy as jnp
from jax import lax
from jax.experimental import pallas as pl
from jax.experimental.pallas import tpu as pltpu
from jax.experimental.pallas import tpu_sc as plsc

_STRIDE = 4
_BS = 32
_NPTS = 17
_H = 128
_W = 128
_FLAT = _H * _W
_NCH = 2 * _NPTS
_PAD = 48
_NCHUNK = _FLAT // 16
_UNROLL = 8
_NBUF = 4


def _lane_shuffle(x, perm):
    return x.at[perm].get(mode="promise_in_bounds")


def _sc_full_body(score_hbm, off_hbm, out_hbm,
                  buf, offidx_v, off_v, out_v, sem0, sem1, sem2, sem3, semg):
    b = lax.axis_index("s") * 2 + lax.axis_index("c")
    base = b * (_NPTS * _FLAT)
    lane = lax.iota(jnp.int32, 16)
    sems = (sem0, sem1, sem2, sem3)
    handles = [None] * _NPTS
    for k in range(_NBUF):
        handles[k] = pltpu.async_copy(
            score_hbm.at[pl.ds(base + k * _FLAT, _FLAT)],
            buf.at[k], sems[k])
    ivs = [lane * 0, lane * 0, lane * 0]
    for r in range(_NPTS):
        handles[r].wait()
        bufr = buf.at[r % _NBUF]

        def chunk_step(c, carry, bufr=bufr):
            rms, chs = carry
            cb = jnp.full((16,), c, jnp.int32)
            new_rms, new_chs = [], []
            for u in range(_UNROLL):
                v = bufr[pl.ds((c * _UNROLL + u) * 16, 16)]
                upd = v > rms[u]
                new_rms.append(jnp.where(upd, v, rms[u]))
                new_chs.append(jnp.where(upd, cb, chs[u]))
            return tuple(new_rms), tuple(new_chs)

        rm0 = jnp.full((16,), -jnp.inf, jnp.float32)
        rms, chs = lax.fori_loop(
            0, _NCHUNK // _UNROLL, chunk_step,
            ((rm0,) * _UNROLL, (lane * 0,) * _UNROLL))
        m = rms[0]
        for u in range(1, _UNROLL):
            m = jnp.maximum(m, rms[u])
        for sh in (8, 4, 2, 1):
            m = jnp.maximum(m, _lane_shuffle(m, lane ^ sh))
        cand = jnp.full((16,), _FLAT, jnp.int32)
        for u in range(_UNROLL):
            flat_u = chs[u] * (16 * _UNROLL) + (u * 16) + lane
            cand = jnp.minimum(cand,
                               jnp.where(rms[u] == m, flat_u, jnp.int32(_FLAT)))
        for sh in (8, 4, 2, 1):
            cand = jnp.minimum(cand, _lane_shuffle(cand, lane ^ sh))
        for g in range(3):
            jv = lane + 16 * g
            ivs[g] = jnp.where((jv >> 1) == r, cand, ivs[g])
        if r + _NBUF < _NPTS:
            handles[r + _NBUF] = pltpu.async_copy(
                score_hbm.at[pl.ds(base + (r + _NBUF) * _FLAT, _FLAT)],
                buf.at[r % _NBUF], sems[r % _NBUF])
    for g in range(3):
        jv = lane + 16 * g
        iv = ivs[g]
        yv = iv >> 7
        xv = iv & (_W - 1)
        oidx = ((b * _NCH + jv) * _H + yv) * _W + xv
        offidx_v[pl.ds(16 * g, 16)] = jnp.where(jv < _NCH, oidx, 0)
    pltpu.async_copy(off_hbm.at[offidx_v], off_v, semg).wait()
    for g in range(3):
        jv = lane + 16 * g
        iv = ivs[g]
        yv = iv >> 7
        xv = iv & (_W - 1)
        coarse = jnp.where((jv & 1) == 0, yv, xv).astype(jnp.float32)
        off = off_v[pl.ds(16 * g, 16)]
        out_v[pl.ds(16 * g, 16)] = (coarse + off) * float(_STRIDE)
    pltpu.sync_copy(out_v, out_hbm.at[pl.ds(b * _PAD, _PAD)])


def _sc_full(score_flat, offset_flat):
    mesh = plsc.VectorSubcoreMesh(core_axis_name="c", subcore_axis_name="s")
    f = functools.partial(
        pl.kernel,
        mesh=mesh,
        out_type=jax.ShapeDtypeStruct((_BS * _PAD,), jnp.float32),
        scratch_types=[
            pltpu.VMEM((_NBUF, _FLAT), jnp.float32),
            pltpu.VMEM((_PAD,), jnp.int32),
            pltpu.VMEM((_PAD,), jnp.float32),
            pltpu.VMEM((_PAD,), jnp.float32),
            pltpu.SemaphoreType.DMA,
            pltpu.SemaphoreType.DMA,
            pltpu.SemaphoreType.DMA,
            pltpu.SemaphoreType.DMA,
            pltpu.SemaphoreType.DMA,
        ],
    )(_sc_full_body)
    return f(score_flat, offset_flat)


def kernel(score_map, offset_map):
    score_flat = score_map.reshape(_BS * _NPTS * _FLAT)
    offset_flat = offset_map.reshape(_BS * _NCH * _FLAT)
    out = _sc_full(score_flat, offset_flat)
    return out.reshape(_BS, _PAD)[:, : _NCH].reshape(_BS, _NPTS, 2)
